# baseline (speedup 1.0000x reference)
import functools

import jax
import jax.numpy as jnp
from jax import lax
from jax.experimental import pallas as pl
from jax.experimental.pallas import tpu as pltpu
from jax.experimental.pallas import tpu_sc as plsc

F = 26
V1 = 100001
E = 16
B = 16384

NC = 2
NS = 16
NW = NC * NS

TOTAL = B * F
BPW = TOTAL // NW
NCHUNK = 8
CH = BPW // NCHUNK

_mesh = plsc.VectorSubcoreMesh(core_axis_name="c", subcore_axis_name="s")


@functools.partial(
    pl.kernel,
    mesh=_mesh,
    out_type=jax.ShapeDtypeStruct((TOTAL, E), jnp.float32),
    scratch_types=[
        pltpu.VMEM((BPW,), jnp.int32),
        pltpu.VMEM((CH, E), jnp.float32),
        pltpu.VMEM((CH, E), jnp.float32),
        pltpu.SemaphoreType.DMA,
        pltpu.SemaphoreType.DMA,
        pltpu.SemaphoreType.DMA,
    ],
    compiler_params=pltpu.CompilerParams(use_tc_tiling_on_sc=False),
)
def _sc_gather(idx_hbm, tab_hbm, out_hbm, idx_v, rows0, rows1, g0sem, g1sem,
               ssem):
    wid = lax.axis_index("s") * NC + lax.axis_index("c")
    base = wid * BPW
    pltpu.sync_copy(idx_hbm.at[pl.ds(base, BPW)], idx_v)

    bufs = (rows0, rows1)
    gsems = (g0sem, g1sem)
    pending = pltpu.async_copy(
        tab_hbm.at[idx_v.at[pl.ds(0, CH)]], bufs[0], gsems[0])
    prev_store = None
    for c in range(NCHUNK):
        if c + 1 < NCHUNK:
            nxt = pltpu.async_copy(
                tab_hbm.at[idx_v.at[pl.ds((c + 1) * CH, CH)]],
                bufs[(c + 1) % 2], gsems[(c + 1) % 2])
        pending.wait()
        if prev_store is not None:
            prev_store.wait()
        prev_store = pltpu.async_copy(
            bufs[c % 2], out_hbm.at[pl.ds(base + c * CH, CH)], ssem)
        if c + 1 < NCHUNK:
            pending = nxt
    prev_store.wait()


def kernel(indices, tables):
    offs = jnp.arange(F, dtype=jnp.int32) * V1
    idx_flat = (indices + offs[None, :]).reshape(TOTAL)
    tab2d = tables.reshape(F * V1, E)
    out = _sc_gather(idx_flat, tab2d)
    return out.reshape(B, F, E)


# speedup vs baseline: 42.3534x; 42.3534x over previous
import functools

import jax
import jax.numpy as jnp
from jax import lax
from jax.experimental import pallas as pl
from jax.experimental.pallas import tpu as pltpu
from jax.experimental.pallas import tpu_sc as plsc

F = 26
V1 = 100001
E = 16
B = 16384

NC = 2
NS = 16
NW = NC * NS

TPW = (F * E) // NW
OCH = 8192
NOCH = B // OCH
UNROLL = 8
GPC = OCH // 16

_mesh = plsc.VectorSubcoreMesh(core_axis_name="c", subcore_axis_name="s")


@functools.partial(
    pl.kernel,
    mesh=_mesh,
    out_type=jax.ShapeDtypeStruct((F, E, B), jnp.float32),
    scratch_types=[
        pltpu.VMEM((V1,), jnp.float32),
        pltpu.VMEM((B,), jnp.int32),
        pltpu.VMEM((OCH,), jnp.float32),
        pltpu.SemaphoreType.DMA,
    ],
    compiler_params=pltpu.CompilerParams(needs_layout_passes=False),
)
def _sc_lookup(idx_hbm, tab_hbm, out_hbm, tab_v, idx_v, out_v, gsem):
    wid = lax.axis_index("s") * NC + lax.axis_index("c")
    for j in range(TPW):
        t = wid * TPW + j
        f = t // E
        e = t % E
        pltpu.sync_copy(idx_hbm.at[f], idx_v)
        pltpu.sync_copy(tab_hbm.at[f, e], tab_v)
        for ch in range(NOCH):

            def body(i, _, ch=ch):
                for u in range(UNROLL):
                    off = i * (16 * UNROLL) + u * 16
                    idx16 = idx_v[pl.ds(ch * OCH + off, 16)]
                    out_v[pl.ds(off, 16)] = plsc.load_gather(tab_v, [idx16])
                return _

            lax.fori_loop(0, GPC // UNROLL, body, None)
            pltpu.sync_copy(out_v, out_hbm.at[f, e, pl.ds(ch * OCH, OCH)])


def kernel(indices, tables):
    idx_t = indices.T
    tab_t = jnp.transpose(tables, (0, 2, 1))
    out_t = _sc_lookup(idx_t, tab_t)
    return jnp.transpose(out_t, (2, 0, 1))


# speedup vs baseline: 64.7597x; 1.5290x over previous
import functools

import jax
import jax.numpy as jnp
from jax import lax
from jax.experimental import pallas as pl
from jax.experimental.pallas import tpu as pltpu
from jax.experimental.pallas import tpu_sc as plsc

F = 26
V1 = 100001
E = 16
B = 16384

NC = 2
NS = 16
NW = NC * NS

TPW = (F * E) // NW
OCH = 4096
NOCH = B // OCH
GPC = OCH // 16

_mesh = plsc.VectorSubcoreMesh(core_axis_name="c", subcore_axis_name="s")


@functools.partial(
    pl.kernel,
    mesh=_mesh,
    out_type=jax.ShapeDtypeStruct((F, E, B), jnp.float32),
    scratch_types=[
        pltpu.VMEM((V1,), jnp.float32),
        pltpu.VMEM((B,), jnp.int32),
        pltpu.VMEM((OCH,), jnp.float32),
        pltpu.VMEM((OCH,), jnp.float32),
        pltpu.SemaphoreType.DMA,
        pltpu.SemaphoreType.DMA,
        pltpu.SemaphoreType.DMA,
    ],
    compiler_params=pltpu.CompilerParams(needs_layout_passes=False),
)
def _sc_lookup(idx_hbm, tab_hbm, out_hbm, tab_v, idx_v, out0, out1,
               tsem, w0sem, w1sem):
    wid = lax.axis_index("s") * NC + lax.axis_index("c")
    obufs = (out0, out1)
    wsems = (w0sem, w1sem)
    prev_wb = [None, None]
    for j in range(TPW):
        t = wid * TPW + j
        f = t // E
        e = t % E
        tcopy = pltpu.async_copy(tab_hbm.at[f, e], tab_v, tsem)
        if j == 0:
            pltpu.sync_copy(idx_hbm.at[f], idx_v)
        else:
            @pl.when(e == 0)
            def _():
                pltpu.sync_copy(idx_hbm.at[f], idx_v)
        tcopy.wait()
        for ch in range(NOCH):
            b = ch % 2
            if prev_wb[b] is not None:
                prev_wb[b].wait()
            ob = obufs[b]

            @plsc.parallel_loop(0, GPC, unroll=8)
            def _gather(i, ch=ch, ob=ob):
                idx16 = idx_v[pl.ds(ch * OCH + i * 16, 16)]
                ob[pl.ds(i * 16, 16)] = plsc.load_gather(tab_v, [idx16])

            prev_wb[b] = pltpu.async_copy(
                ob, out_hbm.at[f, e, pl.ds(ch * OCH, OCH)], wsems[b])
    prev_wb[0].wait()
    prev_wb[1].wait()


def kernel(indices, tables):
    idx_t = indices.T
    tab_t = jnp.transpose(tables, (0, 2, 1))
    out_t = _sc_lookup(idx_t, tab_t)
    return jnp.transpose(out_t, (2, 0, 1))
